# baseline (speedup 1.0000x reference)
import functools

import jax
import jax.numpy as jnp
from jax import lax
from jax.experimental import pallas as pl
from jax.experimental.pallas import tpu as pltpu
from jax.experimental.pallas import tpu_sc as plsc

N_FEATURES = 26
DIM_EMBED = 32
MAX_VALUE = 40000
BATCH = 1024
MAX_L = 50
NP = BATCH * MAX_L
NW = 32
PAIRS_PER_W = NP // NW
C = 64
NCHUNK = PAIRS_PER_W // C
NIDX = C * N_FEATURES
NSTRIP = NIDX // 128
PERIOD = 208
L = 16


def _sc_body(x_hbm, table_hbm, out_hbm, mask_hbm,
             xc0, xc1, idxb0, idxb1, rows0, rows1, outb0, outb1,
             maskb0, maskb1, scaleb0, scaleb1, sem0, sem1, osem, xsem,
             msem):
    wid = lax.axis_index("s") * 2 + lax.axis_index("c")
    iota = lax.iota(jnp.int32, L)

    ovs = [((iota + j * L) % N_FEATURES) * MAX_VALUE
           for j in range(PERIOD // L)]
    gbases = [iota * N_FEATURES + g * L * N_FEATURES
              for g in range(C // L)]
    sem = (sem0, sem1)
    xc = (xc0, xc1)
    idxb = (idxb0, idxb1)
    rows = (rows0, rows1)
    outb = (outb0, outb1)
    maskb = (maskb0, maskb1)
    scaleb = (scaleb0, scaleb1)

    def x_dma(c, par):
        base = c * C + wid * PAIRS_PER_W
        return pltpu.make_async_copy(
            x_hbm.at[pl.ds(base * N_FEATURES, NIDX)], xc[par], xsem)

    def mask_dma(c, par):
        base = c * C + wid * PAIRS_PER_W
        return pltpu.make_async_copy(
            maskb[par], mask_hbm.at[pl.ds(base, C)], msem)

    def strip(par, s):
        return pltpu.make_async_copy(
            table_hbm.at[idxb[par].at[s]],
            rows[par].at[pl.ds(s * 128, 128), :], sem[par])

    def prep(c, par):
        @pl.when(c >= 2)
        def _():
            mask_dma(c, par).wait()
        x_dma(c, par).wait()

        @pl.when(c + 1 < NCHUNK)
        def _():
            x_dma(c + 1, 1 - par).start()
        for j in range(NIDX // L):
            xv = xc[par][pl.ds(j * L, L)]
            idxb[par][j // 8, pl.ds((j % 8) * L, L)] = xv + ovs[j % 13]
            if j % 8 == 7:
                strip(par, j // 8).start()
        for g in range(C // L):
            m = plsc.load_gather(xc[par], [gbases[g]])
            for f in range(1, N_FEATURES):
                m = jnp.maximum(
                    m, plsc.load_gather(xc[par], [gbases[g] + f]))
            is_pad = m == 0
            maskb[par][pl.ds(g * L, L)] = jnp.where(is_pad, 1, 0)
            scaleb[par][pl.ds(g * L, L)] = jnp.where(
                is_pad, 0.0, jnp.float32(1.0 / N_FEATURES))
        mask_dma(c, par).start()

    def drain(par):
        for s in range(NSTRIP):
            strip(par, s).wait()

    def out_dma(c, par):
        base = c * C + wid * PAIRS_PER_W
        return pltpu.make_async_copy(
            outb[par], out_hbm.at[pl.ds(base, C), :], osem)

    def reduce_out(c, par):
        rws = rows[par]
        scb = scaleb[par]

        @pl.when(c >= 2)
        def _():
            out_dma(c, par).wait()

        def red(p2, c2):
            for h in range(2):
                p = p2 * 2 + h
                r0 = p * N_FEATURES
                a0 = rws[r0, pl.ds(0, L)]
                a1 = rws[r0, pl.ds(L, L)]
                for f in range(1, N_FEATURES):
                    a0 = a0 + rws[r0 + f, pl.ds(0, L)]
                    a1 = a1 + rws[r0 + f, pl.ds(L, L)]
                scv = plsc.load_gather(scb, [jnp.broadcast_to(p, (L,))])
                outb[par][p, pl.ds(0, L)] = a0 * scv
                outb[par][p, pl.ds(L, L)] = a1 * scv
            return c2
        lax.fori_loop(0, C // 2, red, 0)
        out_dma(c, par).start()

    x_dma(0, 0).start()
    prep(0, 0)

    def body(i, carry):
        del carry
        for par in range(2):
            c = i * 2 + par
            prep(c + 1, 1 - par)
            drain(par)
            reduce_out(c, par)
        return 0

    lax.fori_loop(0, (NCHUNK - 1) // 2, body, 0)
    drain(0)
    reduce_out(NCHUNK - 1, 0)
    out_dma(NCHUNK - 2, 1).wait()
    out_dma(NCHUNK - 1, 0).wait()
    mask_dma(NCHUNK - 2, 1).wait()
    mask_dma(NCHUNK - 1, 0).wait()


@jax.jit
def _embed(x_flat, table):
    f32 = jnp.float32
    i32 = jnp.int32
    run = functools.partial(
        pl.kernel,
        out_type=[
            jax.ShapeDtypeStruct((NP, DIM_EMBED), f32),
            jax.ShapeDtypeStruct((NP,), i32),
        ],
        mesh=plsc.VectorSubcoreMesh(core_axis_name="c", subcore_axis_name="s"),
        compiler_params=pltpu.CompilerParams(
            needs_layout_passes=False, use_tc_tiling_on_sc=False),
        scratch_types=[
            pltpu.VMEM((NIDX,), i32),
            pltpu.VMEM((NIDX,), i32),
            pltpu.VMEM((NSTRIP, 128), i32),
            pltpu.VMEM((NSTRIP, 128), i32),
            pltpu.VMEM((NIDX, DIM_EMBED), f32),
            pltpu.VMEM((NIDX, DIM_EMBED), f32),
            pltpu.VMEM((C, DIM_EMBED), f32),
            pltpu.VMEM((C, DIM_EMBED), f32),
            pltpu.VMEM((C,), i32),
            pltpu.VMEM((C,), i32),
            pltpu.VMEM((C,), f32),
            pltpu.VMEM((C,), f32),
            pltpu.SemaphoreType.DMA,
            pltpu.SemaphoreType.DMA,
            pltpu.SemaphoreType.DMA,
            pltpu.SemaphoreType.DMA,
            pltpu.SemaphoreType.DMA,
        ],
    )(_sc_body)
    return run(x_flat, table)


def kernel(x, table, ret_first_row):
    del ret_first_row
    x_flat = x.reshape(NP * N_FEATURES)
    out, mask_i = _embed(x_flat, table)
    out = out.reshape(BATCH, MAX_L, DIM_EMBED)
    mask = mask_i.astype(jnp.bool_).reshape(BATCH, MAX_L)
    return (out, mask)


# speedup vs baseline: 1.0090x; 1.0090x over previous
import functools

import jax
import jax.numpy as jnp
from jax import lax
from jax.experimental import pallas as pl
from jax.experimental.pallas import tpu as pltpu
from jax.experimental.pallas import tpu_sc as plsc

N_FEATURES = 26
DIM_EMBED = 32
MAX_VALUE = 40000
BATCH = 1024
MAX_L = 50
NP = BATCH * MAX_L
NW = 32
PAIRS_PER_W = NP // NW
C = 64
NCHUNK = PAIRS_PER_W // C
NIDX = C * N_FEATURES
NSTRIP = NIDX // 128
PERIOD = 208
L = 16


def _sc_body(x_hbm, table_hbm, out_hbm, mask_hbm,
             xc0, xc1, idxb0, idxb1, rows0, rows1, outb0, outb1,
             maskb0, maskb1, scaleb0, scaleb1, sem0, sem1, osem, xsem,
             msem):
    wid = lax.axis_index("s") * 2 + lax.axis_index("c")
    iota = lax.iota(jnp.int32, L)

    ovs = [((iota + j * L) % N_FEATURES) * MAX_VALUE
           for j in range(PERIOD // L)]
    gbases = [iota * N_FEATURES + g * L * N_FEATURES
              for g in range(C // L)]
    sem = (sem0, sem1)
    xc = (xc0, xc1)
    idxb = (idxb0, idxb1)
    rows = (rows0, rows1)
    outb = (outb0, outb1)
    maskb = (maskb0, maskb1)
    scaleb = (scaleb0, scaleb1)

    def x_dma(c, par):
        base = c * C + wid * PAIRS_PER_W
        return pltpu.make_async_copy(
            x_hbm.at[pl.ds(base * N_FEATURES, NIDX)], xc[par], xsem)

    def mask_dma(c, par):
        base = c * C + wid * PAIRS_PER_W
        return pltpu.make_async_copy(
            maskb[par], mask_hbm.at[pl.ds(base, C)], msem)

    def strip(par, s):
        return pltpu.make_async_copy(
            table_hbm.at[idxb[par].at[s]],
            rows[par].at[pl.ds(s * 128, 128), :], sem[par])

    def prep(c, par):
        @pl.when(c >= 2)
        def _():
            mask_dma(c, par).wait()
        x_dma(c, par).wait()

        @pl.when(c + 1 < NCHUNK)
        def _():
            x_dma(c + 1, 1 - par).start()
        for j in range(NIDX // L):
            xv = xc[par][pl.ds(j * L, L)]
            idxb[par][j // 8, pl.ds((j % 8) * L, L)] = xv + ovs[j % 13]
        for s in range(NSTRIP):
            strip(par, s).start()
        for g in range(C // L):
            m = plsc.load_gather(xc[par], [gbases[g]])
            for f in range(1, N_FEATURES):
                m = jnp.maximum(
                    m, plsc.load_gather(xc[par], [gbases[g] + f]))
            is_pad = m == 0
            maskb[par][pl.ds(g * L, L)] = jnp.where(is_pad, 1, 0)
            scaleb[par][pl.ds(g * L, L)] = jnp.where(
                is_pad, 0.0, jnp.float32(1.0 / N_FEATURES))
        mask_dma(c, par).start()

    def drain(par):
        for s in range(NSTRIP):
            strip(par, s).wait()

    def out_dma(c, par):
        base = c * C + wid * PAIRS_PER_W
        return pltpu.make_async_copy(
            outb[par], out_hbm.at[pl.ds(base, C), :], osem)

    def reduce_out(c, par):
        rws = rows[par]
        scb = scaleb[par]

        @pl.when(c >= 2)
        def _():
            out_dma(c, par).wait()

        def red(p2, c2):
            for h in range(2):
                p = p2 * 2 + h
                r0 = p * N_FEATURES
                a0 = rws[r0, pl.ds(0, L)]
                a1 = rws[r0, pl.ds(L, L)]
                for f in range(1, N_FEATURES):
                    a0 = a0 + rws[r0 + f, pl.ds(0, L)]
                    a1 = a1 + rws[r0 + f, pl.ds(L, L)]
                scv = plsc.load_gather(scb, [jnp.broadcast_to(p, (L,))])
                outb[par][p, pl.ds(0, L)] = a0 * scv
                outb[par][p, pl.ds(L, L)] = a1 * scv
            return c2
        lax.fori_loop(0, C // 2, red, 0)
        out_dma(c, par).start()

    x_dma(0, 0).start()
    prep(0, 0)

    def body(i, carry):
        del carry
        for par in range(2):
            c = i * 2 + par
            prep(c + 1, 1 - par)
            drain(par)
            reduce_out(c, par)
        return 0

    lax.fori_loop(0, (NCHUNK - 1) // 2, body, 0)
    drain(0)
    reduce_out(NCHUNK - 1, 0)
    out_dma(NCHUNK - 2, 1).wait()
    out_dma(NCHUNK - 1, 0).wait()
    mask_dma(NCHUNK - 2, 1).wait()
    mask_dma(NCHUNK - 1, 0).wait()


@jax.jit
def _embed(x_flat, table):
    f32 = jnp.float32
    i32 = jnp.int32
    run = functools.partial(
        pl.kernel,
        out_type=[
            jax.ShapeDtypeStruct((NP, DIM_EMBED), f32),
            jax.ShapeDtypeStruct((NP,), i32),
        ],
        mesh=plsc.VectorSubcoreMesh(core_axis_name="c", subcore_axis_name="s"),
        compiler_params=pltpu.CompilerParams(
            needs_layout_passes=False, use_tc_tiling_on_sc=False),
        scratch_types=[
            pltpu.VMEM((NIDX,), i32),
            pltpu.VMEM((NIDX,), i32),
            pltpu.VMEM((NSTRIP, 128), i32),
            pltpu.VMEM((NSTRIP, 128), i32),
            pltpu.VMEM((NIDX, DIM_EMBED), f32),
            pltpu.VMEM((NIDX, DIM_EMBED), f32),
            pltpu.VMEM((C, DIM_EMBED), f32),
            pltpu.VMEM((C, DIM_EMBED), f32),
            pltpu.VMEM((C,), i32),
            pltpu.VMEM((C,), i32),
            pltpu.VMEM((C,), f32),
            pltpu.VMEM((C,), f32),
            pltpu.SemaphoreType.DMA,
            pltpu.SemaphoreType.DMA,
            pltpu.SemaphoreType.DMA,
            pltpu.SemaphoreType.DMA,
            pltpu.SemaphoreType.DMA,
        ],
    )(_sc_body)
    return run(x_flat, table)


def kernel(x, table, ret_first_row):
    del ret_first_row
    x_flat = x.reshape(NP * N_FEATURES)
    out, mask_i = _embed(x_flat, table)
    out = out.reshape(BATCH, MAX_L, DIM_EMBED)
    mask = mask_i.astype(jnp.bool_).reshape(BATCH, MAX_L)
    return (out, mask)
